# Initial kernel scaffold; baseline (speedup 1.0000x reference)
import jax
import jax.numpy as jnp
from jax.experimental import pallas as pl


def kernel(x, rrwp_index, rrwp_conn, WQ, WK, WV, WEw, WEb, bEb, WEo, bEo, Aw, BW):
    raise NotImplementedError("write your pallas kernel here")


# speedup vs baseline: 29.8151x; 29.8151x over previous
import functools

import jax
import jax.numpy as jnp
from jax import lax
from jax.experimental import pallas as pl
from jax.experimental.pallas import tpu as pltpu
from jax.experimental.pallas import tpu_sc as plsc

N = 10000
NPAD = 10240
E = 320000
HID = 128
HEADS = 8
DIM = 8
DI = HEADS * DIM
CLAMP = 5.0

NC = 2
NS = 16
NW = NC * NS
EPW = E // NW
TPD = NPAD // NS
NF = NPAD * HEADS

_MESH = dict(core_axis_name="c", subcore_axis_name="s", num_cores=NC,
             num_subcores=NS)


def _dot(a, b):
    return lax.dot_general(a, b, (((1,), (0,)), ((), ())),
                           precision=lax.Precision.HIGHEST,
                           preferred_element_type=jnp.float32)


def _qkv_body(x_ref, w3_ref, qk_ref, vp_ref):
    prod = _dot(x_ref[...], w3_ref[...])
    qk_ref[...] = prod[:, :2 * DI]
    vp_ref[...] = jnp.concatenate(
        [prod[:, 2 * DI:], jnp.zeros((N, DI), jnp.float32)], axis=1)


def _qkv(x, w3):
    return pl.pallas_call(
        _qkv_body,
        out_shape=[jax.ShapeDtypeStruct((N, HID), jnp.float32)] * 2,
    )(x, w3)


_CH1 = 80


def _msg_kernel(qk, dst, src, out, idxd_v, idxs_v, gq, gk, m_v):
    wid = lax.axis_index("s") * NC + lax.axis_index("c")
    base = wid * EPW

    def chunk(i, carry):
        off = base + i * _CH1
        pltpu.sync_copy(dst.at[pl.ds(off, _CH1)], idxd_v)
        pltpu.sync_copy(src.at[pl.ds(off, _CH1)], idxs_v)
        pltpu.sync_copy(qk.at[idxd_v], gq)
        pltpu.sync_copy(qk.at[idxs_v], gk)

        def add_row(r, c2):
            for c in range(DI // 16):
                s = pl.ds(c * 16, 16)
                s2 = pl.ds(DI + c * 16, 16)
                m_v[r, s] = gq[r, s] + gk[r, s2]
            return c2

        lax.fori_loop(0, _CH1, add_row, 0)
        pltpu.sync_copy(m_v, out.at[pl.ds(off, _CH1)])
        return carry

    lax.fori_loop(0, EPW // _CH1, chunk, 0)


@functools.lru_cache(maxsize=None)
def _get_msg_call():
    return functools.partial(
        pl.kernel,
        out_type=jax.ShapeDtypeStruct((E, DI), jnp.float32),
        mesh=plsc.VectorSubcoreMesh(**_MESH),
        compiler_params=pltpu.CompilerParams(needs_layout_passes=False),
        scratch_types=[
            pltpu.VMEM((_CH1,), jnp.int32),
            pltpu.VMEM((_CH1,), jnp.int32),
            pltpu.VMEM((_CH1, HID), jnp.float32),
            pltpu.VMEM((_CH1, HID), jnp.float32),
            pltpu.VMEM((_CH1, DI), jnp.float32),
        ],
    )(_msg_kernel)


_BE = 2000


def _edge_body(cb_ref, m_ref, wEwb_ref, bEb_ref, wEoA_ref, bEoA_ref,
               oe_ref, ex_ref):
    big = _dot(cb_ref[...], wEwb_ref[...])
    ew = big[:, :DI]
    eb = big[:, DI:] + bEb_ref[...]
    c1 = m_ref[...] * ew
    c2 = jnp.sign(c1) * jnp.sqrt(jnp.abs(c1))
    c3 = jnp.maximum(c2 + eb, 0.0)
    out2 = _dot(c3, wEoA_ref[...]) + bEoA_ref[...]
    oe_ref[...] = out2[:, :DI]
    sc = jnp.clip(out2[:, DI:], -CLAMP, CLAMP)
    ex_ref[...] = jnp.exp(sc)


def _edge(cb, m, wEwb, bEb2, wEoA, bEoA2):
    nblk = E // _BE
    return pl.pallas_call(
        _edge_body,
        grid=(nblk,),
        in_specs=[
            pl.BlockSpec((_BE, HID), lambda i: (i, 0)),
            pl.BlockSpec((_BE, DI), lambda i: (i, 0)),
            pl.BlockSpec((HID, 2 * DI), lambda i: (0, 0)),
            pl.BlockSpec((1, DI), lambda i: (0, 0)),
            pl.BlockSpec((DI, DI + HEADS), lambda i: (0, 0)),
            pl.BlockSpec((1, DI + HEADS), lambda i: (0, 0)),
        ],
        out_specs=[
            pl.BlockSpec((_BE, DI), lambda i: (i, 0)),
            pl.BlockSpec((_BE, HEADS), lambda i: (i, 0)),
        ],
        out_shape=[
            jax.ShapeDtypeStruct((E, DI), jnp.float32),
            jax.ShapeDtypeStruct((E, HEADS), jnp.float32),
        ],
    )(cb, m, wEwb, bEb2, wEoA, bEoA2)


_CH3 = 1000


def _ssum_kernel(dst, exf, out, idx_v, ex_v, acc_v):
    cid = lax.axis_index("c")
    sid = lax.axis_index("s")
    wid = sid * NC + cid
    zero16 = jnp.zeros((16,), jnp.float32)
    iota = lax.iota(jnp.int32, 16)
    mask8 = iota < 8

    def zstep(j, carry):
        acc_v[pl.ds(j * 16, 16)] = zero16
        return carry

    lax.fori_loop(0, NF // 16, zstep, 0)
    base = wid * EPW

    def chunk(i, carry):
        off = base + i * _CH3
        pltpu.sync_copy(dst.at[pl.ds(off, _CH3)], idx_v)
        pltpu.sync_copy(exf.at[pl.ds(off * HEADS, _CH3 * HEADS)],
                        ex_v.at[pl.ds(0, _CH3 * HEADS)])

        def estep(j, c2):
            d16 = plsc.load_gather(idx_v, [jnp.full((16,), j, jnp.int32)])
            a16 = d16 * HEADS + iota
            x16 = ex_v[pl.ds(j * HEADS, 16)]
            plsc.addupdate_scatter(acc_v, [a16], x16, mask=mask8)
            return c2

        lax.fori_loop(0, _CH3, estep, 0)
        return carry

    lax.fori_loop(0, EPW // _CH3, chunk, 0)
    pltpu.sync_copy(acc_v, out.at[wid])


@functools.lru_cache(maxsize=None)
def _get_ssum_call():
    return functools.partial(
        pl.kernel,
        out_type=jax.ShapeDtypeStruct((NW, NF), jnp.float32),
        mesh=plsc.VectorSubcoreMesh(**_MESH),
        compiler_params=pltpu.CompilerParams(needs_layout_passes=False),
        scratch_types=[
            pltpu.VMEM((_CH3,), jnp.int32),
            pltpu.VMEM((_CH3 * HEADS + 16,), jnp.float32),
            pltpu.VMEM((NF,), jnp.float32),
        ],
    )(_ssum_kernel)


def _recip_body(pp_ref, p2_ref, out_ref):
    s = jnp.sum(pp_ref[...], axis=0)
    out_ref[...] = _dot(1.0 / (s + 1e-16), p2_ref[...])


def _recip(pp, p2):
    return pl.pallas_call(
        _recip_body,
        out_shape=jax.ShapeDtypeStruct((NF // HID, 16 * HID), jnp.float32),
    )(pp, p2)


_CH4 = 80


def _agg_kernel(dst, src, exf, oe, rsum, vp, zz, out,
                idxd_v, idxs_v, ex_v, vs_v, oe_v, ct_v, rg_v, acc_sh):
    cid = lax.axis_index("c")
    sid = lax.axis_index("s")
    wid = sid * NC + cid
    rslice = pl.ds(sid * TPD, TPD)
    pltpu.sync_copy(zz.at[rslice], acc_sh.at[rslice])
    plsc.subcore_barrier()
    iota = lax.iota(jnp.int32, 16)
    half = iota // HEADS
    base = wid * EPW

    def chunk(i, carry):
        off = base + i * _CH4
        pltpu.sync_copy(dst.at[pl.ds(off, _CH4)], idxd_v)
        pltpu.sync_copy(src.at[pl.ds(off, _CH4)], idxs_v)
        pltpu.sync_copy(vp.at[idxs_v], vs_v)
        pltpu.sync_copy(rsum.at[idxd_v], rg_v)
        pltpu.sync_copy(oe.at[pl.ds(off, _CH4)], oe_v)
        pltpu.sync_copy(exf.at[pl.ds(off * HEADS, _CH4 * HEADS)],
                        ex_v.at[pl.ds(0, _CH4 * HEADS)])

        def row(r, c2):
            e16 = jnp.full((16,), r * HEADS, jnp.int32)
            for c in range(DI // 16):
                pat = c * 2 + half
                eg = plsc.load_gather(ex_v, [e16 + pat])
                s = pl.ds(c * 16, 16)
                s2 = pl.ds(DI + c * 16, 16)
                w = eg * rg_v[r, s]
                ct_v[r, s] = vs_v[r, s] * w
                ct_v[r, s2] = oe_v[r, s] * w
            return c2

        lax.fori_loop(0, _CH4, row, 0)
        pltpu.sync_copy(ct_v, acc_sh.at[idxd_v], add=True)
        return carry

    lax.fori_loop(0, EPW // _CH4, chunk, 0)
    plsc.subcore_barrier()
    pltpu.sync_copy(acc_sh.at[rslice], out.at[cid, rslice])


@functools.lru_cache(maxsize=None)
def _get_agg_call():
    return functools.partial(
        pl.kernel,
        out_type=jax.ShapeDtypeStruct((NC, NPAD, 2 * DI), jnp.float32),
        mesh=plsc.VectorSubcoreMesh(**_MESH),
        compiler_params=pltpu.CompilerParams(needs_layout_passes=False),
        scratch_types=[
            pltpu.VMEM((_CH4,), jnp.int32),
            pltpu.VMEM((_CH4,), jnp.int32),
            pltpu.VMEM((_CH4 * HEADS + 16,), jnp.float32),
            pltpu.VMEM((_CH4, HID), jnp.float32),
            pltpu.VMEM((_CH4, DI), jnp.float32),
            pltpu.VMEM((_CH4, 2 * DI), jnp.float32),
            pltpu.VMEM((_CH4, HID), jnp.float32),
            pltpu.VMEM_SHARED((NPAD, 2 * DI), jnp.float32),
        ],
    )(_agg_kernel)


def _final_body(p_ref, b2_ref, out_ref):
    s = p_ref[0] + p_ref[1]
    sn = s[:N]
    out_ref[...] = sn[:, :DI] + _dot(sn[:, DI:], b2_ref[...])


def _final(parts, b2):
    return pl.pallas_call(
        _final_body,
        out_shape=jax.ShapeDtypeStruct((N, DI), jnp.float32),
    )(parts, b2)


def kernel(x, rrwp_index, rrwp_conn, WQ, WK, WV, WEw, WEb, bEb, WEo, bEo,
           Aw, BW):
    f32 = jnp.float32
    dst = rrwp_index[0].astype(jnp.int32)
    src = rrwp_index[1].astype(jnp.int32)

    w3 = jnp.concatenate([WQ, WK, WV], axis=1)
    wEwb = jnp.concatenate([WEw, WEb], axis=1)
    bEb2 = bEb.reshape(1, DI)
    i64 = jnp.arange(DI)
    A2 = jnp.zeros((DI, HEADS), f32).at[i64, i64 // DIM].set(
        Aw[:, :, 0].T.reshape(DI))
    wEoA = jnp.concatenate([WEo, WEo @ A2], axis=1)
    bEoA = jnp.concatenate([bEo, bEo @ A2]).reshape(1, DI + HEADS)
    B2 = jnp.zeros((DI, DI), f32).at[
        i64[:, None],
        (i64[:, None] // DIM) * DIM + jnp.arange(DIM)[None, :]].set(
        BW.transpose(1, 0, 2).reshape(DI, DIM))

    zz128 = jnp.zeros((NPAD, 2 * DI), f32)

    qk, vp = _qkv(x, w3)
    msg1 = _get_msg_call()(qk, dst, src)
    oe, ex8 = _edge(rrwp_conn, msg1, wEwb, bEb2, wEoA, bEoA)
    exf = ex8.reshape(E * HEADS)
    a2048 = jnp.arange(16 * HID)
    P2 = jnp.zeros((HID, 16 * HID), f32).at[
        HEADS * (a2048 // HID) + (a2048 % HID) // HEADS, a2048].set(1.0)
    ssum_p = _get_ssum_call()(dst, exf)
    rsum = _recip(ssum_p.reshape(NW, NF // HID, HID), P2).reshape(NPAD, HID)
    parts = _get_agg_call()(dst, src, exf, oe, rsum, vp, zz128)
    h_out = _final(parts, B2)
    return (h_out, oe)


# speedup vs baseline: 32.3060x; 1.0835x over previous
import functools

import jax
import jax.numpy as jnp
from jax import lax
from jax.experimental import pallas as pl
from jax.experimental.pallas import tpu as pltpu
from jax.experimental.pallas import tpu_sc as plsc

N = 10000
NPAD = 10240
E = 320000
HID = 128
HEADS = 8
DIM = 8
DI = HEADS * DIM
CLAMP = 5.0

NC = 2
NS = 16
NW = NC * NS
EPW = E // NW
TPD = NPAD // NS
NF = NPAD * HEADS

_MESH = dict(core_axis_name="c", subcore_axis_name="s", num_cores=NC,
             num_subcores=NS)


def _dot(a, b):
    return lax.dot_general(a, b, (((1,), (0,)), ((), ())),
                           precision=lax.Precision.HIGHEST,
                           preferred_element_type=jnp.float32)


def _qkv_body(x_ref, w3_ref, qk_ref, vp_ref):
    prod = _dot(x_ref[...], w3_ref[...])
    qk_ref[...] = prod[:, :2 * DI]
    vp_ref[...] = jnp.concatenate(
        [prod[:, 2 * DI:], jnp.zeros((N, DI), jnp.float32)], axis=1)


def _qkv(x, w3):
    return pl.pallas_call(
        _qkv_body,
        out_shape=[jax.ShapeDtypeStruct((N, HID), jnp.float32)] * 2,
    )(x, w3)


_CH1 = 80


def _msg_kernel(qk, dst, src, out, idxd_v, idxs_v, gq, gk, m_v):
    wid = lax.axis_index("s") * NC + lax.axis_index("c")
    base = wid * EPW

    def chunk(i, carry):
        off = base + i * _CH1
        pltpu.sync_copy(dst.at[pl.ds(off, _CH1)], idxd_v)
        pltpu.sync_copy(src.at[pl.ds(off, _CH1)], idxs_v)
        pltpu.sync_copy(qk.at[idxd_v], gq)
        pltpu.sync_copy(qk.at[idxs_v], gk)

        def add_row(r, c2):
            for c in range(DI // 16):
                ss = pl.ds(c * 16, 16)
                s2 = pl.ds(DI + c * 16, 16)
                m_v[r, ss] = gq[r, ss] + gk[r, s2]
            return c2

        lax.fori_loop(0, _CH1, add_row, 0)
        pltpu.sync_copy(m_v, out.at[pl.ds(off, _CH1)])
        return carry

    lax.fori_loop(0, EPW // _CH1, chunk, 0)


@functools.lru_cache(maxsize=None)
def _get_msg_call():
    return functools.partial(
        pl.kernel,
        out_type=jax.ShapeDtypeStruct((E, DI), jnp.float32),
        mesh=plsc.VectorSubcoreMesh(**_MESH),
        compiler_params=pltpu.CompilerParams(needs_layout_passes=False),
        scratch_types=[
            pltpu.VMEM((_CH1,), jnp.int32),
            pltpu.VMEM((_CH1,), jnp.int32),
            pltpu.VMEM((_CH1, HID), jnp.float32),
            pltpu.VMEM((_CH1, HID), jnp.float32),
            pltpu.VMEM((_CH1, DI), jnp.float32),
        ],
    )(_msg_kernel)


_BE = 2000


def _edge_body(cb_ref, m_ref, wEwb_ref, bEb_ref, wEoA_ref, bEoA_ref,
               oe_ref, ex_ref):
    big = _dot(cb_ref[...], wEwb_ref[...])
    ew = big[:, :DI]
    eb = big[:, DI:] + bEb_ref[...]
    c1 = m_ref[...] * ew
    c2 = jnp.sign(c1) * jnp.sqrt(jnp.abs(c1))
    c3 = jnp.maximum(c2 + eb, 0.0)
    out2 = _dot(c3, wEoA_ref[...]) + bEoA_ref[...]
    oe_ref[...] = out2[:, :DI]
    sc = jnp.clip(out2[:, DI:], -CLAMP, CLAMP)
    ex_ref[...] = jnp.exp(sc)


def _edge(cb, m, wEwb, bEb2, wEoA, bEoA2):
    nblk = E // _BE
    return pl.pallas_call(
        _edge_body,
        grid=(nblk,),
        in_specs=[
            pl.BlockSpec((_BE, HID), lambda i: (i, 0)),
            pl.BlockSpec((_BE, DI), lambda i: (i, 0)),
            pl.BlockSpec((HID, 2 * DI), lambda i: (0, 0)),
            pl.BlockSpec((1, DI), lambda i: (0, 0)),
            pl.BlockSpec((DI, DI + HEADS), lambda i: (0, 0)),
            pl.BlockSpec((1, DI + HEADS), lambda i: (0, 0)),
        ],
        out_specs=[
            pl.BlockSpec((_BE, DI), lambda i: (i, 0)),
            pl.BlockSpec((_BE, HEADS), lambda i: (i, 0)),
        ],
        out_shape=[
            jax.ShapeDtypeStruct((E, DI), jnp.float32),
            jax.ShapeDtypeStruct((E, HEADS), jnp.float32),
        ],
    )(cb, m, wEwb, bEb2, wEoA, bEoA2)


_CH3 = 1000


def _ssum_kernel(dst, exf, out, idx_v, ex_v, acc_v):
    cid = lax.axis_index("c")
    sid = lax.axis_index("s")
    wid = sid * NC + cid
    zero16 = jnp.zeros((16,), jnp.float32)
    iota = lax.iota(jnp.int32, 16)
    mask8 = iota < 8

    def zstep(j, carry):
        acc_v[pl.ds(j * 16, 16)] = zero16
        return carry

    lax.fori_loop(0, NF // 16, zstep, 0)
    base = wid * EPW

    def chunk(i, carry):
        off = base + i * _CH3
        pltpu.sync_copy(dst.at[pl.ds(off, _CH3)], idx_v)
        pltpu.sync_copy(exf.at[pl.ds(off * HEADS, _CH3 * HEADS)],
                        ex_v.at[pl.ds(0, _CH3 * HEADS)])

        def estep(j, c2):
            d16 = plsc.load_gather(idx_v, [jnp.full((16,), j, jnp.int32)])
            a16 = d16 * HEADS + iota
            x16 = ex_v[pl.ds(j * HEADS, 16)]
            plsc.addupdate_scatter(acc_v, [a16], x16, mask=mask8)
            return c2

        lax.fori_loop(0, _CH3, estep, 0)
        return carry

    lax.fori_loop(0, EPW // _CH3, chunk, 0)
    pltpu.sync_copy(acc_v, out.at[wid])


@functools.lru_cache(maxsize=None)
def _get_ssum_call():
    return functools.partial(
        pl.kernel,
        out_type=jax.ShapeDtypeStruct((NW, NF), jnp.float32),
        mesh=plsc.VectorSubcoreMesh(**_MESH),
        compiler_params=pltpu.CompilerParams(needs_layout_passes=False),
        scratch_types=[
            pltpu.VMEM((_CH3,), jnp.int32),
            pltpu.VMEM((_CH3 * HEADS + 16,), jnp.float32),
            pltpu.VMEM((NF,), jnp.float32),
        ],
    )(_ssum_kernel)


def _recip_body(pp_ref, p2_ref, out_ref):
    s = jnp.sum(pp_ref[...], axis=0)
    out_ref[...] = _dot(1.0 / (s + 1e-16), p2_ref[...])


def _recip(pp, p2):
    return pl.pallas_call(
        _recip_body,
        out_shape=jax.ShapeDtypeStruct((NF // HID, 16 * HID), jnp.float32),
    )(pp, p2)


_CH4 = 80


def _agg_kernel(dst, src, exf, oe, vp, zz, out,
                idxd_v, idxs_v, ex_v, oe_v, ct_v, acc_sh):
    cid = lax.axis_index("c")
    sid = lax.axis_index("s")
    wid = sid * NC + cid
    rslice = pl.ds(sid * TPD, TPD)
    pltpu.sync_copy(zz.at[rslice], acc_sh.at[rslice])
    plsc.subcore_barrier()
    iota = lax.iota(jnp.int32, 16)
    half = iota // HEADS
    base = wid * EPW

    def chunk(i, carry):
        off = base + i * _CH4
        pltpu.sync_copy(dst.at[pl.ds(off, _CH4)], idxd_v)
        pltpu.sync_copy(src.at[pl.ds(off, _CH4)], idxs_v)
        pltpu.sync_copy(vp.at[idxs_v], ct_v)
        pltpu.sync_copy(oe.at[pl.ds(off, _CH4)], oe_v)
        pltpu.sync_copy(exf.at[pl.ds(off * HEADS, _CH4 * HEADS)],
                        ex_v.at[pl.ds(0, _CH4 * HEADS)])

        def row(r, c2):
            e16 = jnp.full((16,), r * HEADS, jnp.int32)
            for cc in range(DI // 16):
                w = plsc.load_gather(ex_v, [e16 + (cc * 2 + half)])
                ss = pl.ds(cc * 16, 16)
                s2 = pl.ds(DI + cc * 16, 16)
                ct_v[r, ss] = ct_v[r, ss] * w
                ct_v[r, s2] = oe_v[r, ss] * w
            return c2

        lax.fori_loop(0, _CH4, row, 0)
        pltpu.sync_copy(ct_v, acc_sh.at[idxd_v], add=True)
        return carry

    lax.fori_loop(0, EPW // _CH4, chunk, 0)
    plsc.subcore_barrier()
    pltpu.sync_copy(acc_sh.at[rslice], out.at[cid, rslice])


@functools.lru_cache(maxsize=None)
def _get_agg_call():
    return functools.partial(
        pl.kernel,
        out_type=jax.ShapeDtypeStruct((NC, NPAD, 2 * DI), jnp.float32),
        mesh=plsc.VectorSubcoreMesh(**_MESH),
        compiler_params=pltpu.CompilerParams(needs_layout_passes=False),
        scratch_types=[
            pltpu.VMEM((_CH4,), jnp.int32),
            pltpu.VMEM((_CH4,), jnp.int32),
            pltpu.VMEM((_CH4 * HEADS + 16,), jnp.float32),
            pltpu.VMEM((_CH4, DI), jnp.float32),
            pltpu.VMEM((_CH4, 2 * DI), jnp.float32),
            pltpu.VMEM_SHARED((NPAD, 2 * DI), jnp.float32),
        ],
    )(_agg_kernel)


def _final_body(p_ref, r_ref, b2_ref, out_ref):
    s = (p_ref[0] + p_ref[1]) * r_ref[...]
    sn = s[:N]
    out_ref[...] = sn[:, :DI] + _dot(sn[:, DI:], b2_ref[...])


def _final(parts, rsum, b2):
    return pl.pallas_call(
        _final_body,
        out_shape=jax.ShapeDtypeStruct((N, DI), jnp.float32),
    )(parts, rsum, b2)


def kernel(x, rrwp_index, rrwp_conn, WQ, WK, WV, WEw, WEb, bEb, WEo, bEo,
           Aw, BW):
    f32 = jnp.float32
    dst = rrwp_index[0].astype(jnp.int32)
    src = rrwp_index[1].astype(jnp.int32)

    w3 = jnp.concatenate([WQ, WK, WV], axis=1)
    wEwb = jnp.concatenate([WEw, WEb], axis=1)
    bEb2 = bEb.reshape(1, DI)
    i64 = jnp.arange(DI)
    A2 = jnp.zeros((DI, HEADS), f32).at[i64, i64 // DIM].set(
        Aw[:, :, 0].T.reshape(DI))
    wEoA = jnp.concatenate([WEo, WEo @ A2], axis=1)
    bEoA = jnp.concatenate([bEo, bEo @ A2]).reshape(1, DI + HEADS)
    B2 = jnp.zeros((DI, DI), f32).at[
        i64[:, None],
        (i64[:, None] // DIM) * DIM + jnp.arange(DIM)[None, :]].set(
        BW.transpose(1, 0, 2).reshape(DI, DIM))

    zz128 = jnp.zeros((NPAD, 2 * DI), f32)

    qk, vp = _qkv(x, w3)
    msg1 = _get_msg_call()(qk, dst, src)
    oe, ex8 = _edge(rrwp_conn, msg1, wEwb, bEb2, wEoA, bEoA)
    exf = ex8.reshape(E * HEADS)
    a2048 = jnp.arange(16 * HID)
    P2 = jnp.zeros((HID, 16 * HID), f32).at[
        HEADS * (a2048 // HID) + ((a2048 % HID) % DI) // HEADS, a2048].set(1.0)
    ssum_p = _get_ssum_call()(dst, exf)
    rsum = _recip(ssum_p.reshape(NW, NF // HID, HID), P2).reshape(NPAD, HID)
    parts = _get_agg_call()(dst, src, exf, oe, vp, zz128)
    h_out = _final(parts, rsum, B2)
    return (h_out, oe)


# speedup vs baseline: 38.4466x; 1.1901x over previous
import functools

import jax
import jax.numpy as jnp
from jax import lax
from jax.experimental import pallas as pl
from jax.experimental.pallas import tpu as pltpu
from jax.experimental.pallas import tpu_sc as plsc

N = 10000
NPAD = 10240
E = 320000
HID = 128
HEADS = 8
DIM = 8
DI = HEADS * DIM
CLAMP = 5.0

NC = 2
NS = 16
NW = NC * NS
EPW = E // NW
TPD = NPAD // NS
NF = NPAD * HEADS

_MESH = dict(core_axis_name="c", subcore_axis_name="s", num_cores=NC,
             num_subcores=NS)


def _dot(a, b):
    return lax.dot_general(a, b, (((1,), (0,)), ((), ())),
                           precision=lax.Precision.HIGHEST,
                           preferred_element_type=jnp.float32)


def _qkv_body(x_ref, w3_ref, qk_ref, vp_ref):
    prod = _dot(x_ref[...], w3_ref[...])
    qk_ref[...] = prod[:, :2 * DI]
    vp_ref[...] = jnp.concatenate(
        [prod[:, 2 * DI:], jnp.zeros((N, DI), jnp.float32)], axis=1)


def _qkv(x, w3):
    return pl.pallas_call(
        _qkv_body,
        out_shape=[jax.ShapeDtypeStruct((N, HID), jnp.float32)] * 2,
    )(x, w3)


_CH1 = 80


def _msg_kernel(qk, dst, src, out, idxd_v, idxs_v, gq, gk, m_v, sem1, sem2):
    wid = lax.axis_index("s") * NC + lax.axis_index("c")
    base = wid * EPW

    def chunk(i, carry):
        off = base + i * _CH1
        d1 = pltpu.async_copy(dst.at[pl.ds(off, _CH1)], idxd_v, sem1)
        d2 = pltpu.async_copy(src.at[pl.ds(off, _CH1)], idxs_v, sem2)
        d1.wait()
        d2.wait()
        d3 = pltpu.async_copy(qk.at[idxd_v], gq, sem1)
        d4 = pltpu.async_copy(qk.at[idxs_v], gk, sem2)
        d3.wait()
        d4.wait()

        def add_row(r, c2):
            for c in range(DI // 16):
                ss = pl.ds(c * 16, 16)
                s2 = pl.ds(DI + c * 16, 16)
                m_v[r, ss] = gq[r, ss] + gk[r, s2]
            return c2

        lax.fori_loop(0, _CH1, add_row, 0)
        pltpu.sync_copy(m_v, out.at[pl.ds(off, _CH1)])
        return carry

    lax.fori_loop(0, EPW // _CH1, chunk, 0)


@functools.lru_cache(maxsize=None)
def _get_msg_call():
    return functools.partial(
        pl.kernel,
        out_type=jax.ShapeDtypeStruct((E, DI), jnp.float32),
        mesh=plsc.VectorSubcoreMesh(**_MESH),
        compiler_params=pltpu.CompilerParams(needs_layout_passes=False),
        scratch_types=[
            pltpu.VMEM((_CH1,), jnp.int32),
            pltpu.VMEM((_CH1,), jnp.int32),
            pltpu.VMEM((_CH1, HID), jnp.float32),
            pltpu.VMEM((_CH1, HID), jnp.float32),
            pltpu.VMEM((_CH1, DI), jnp.float32),
            pltpu.SemaphoreType.DMA,
            pltpu.SemaphoreType.DMA,
        ],
    )(_msg_kernel)


_BE = 2000


def _edge_body(cb_ref, m_ref, wEwb_ref, bEb_ref, wEoA_ref, bEoA_ref,
               oe_ref, ex_ref):
    big = _dot(cb_ref[...], wEwb_ref[...])
    ew = big[:, :DI]
    eb = big[:, DI:] + bEb_ref[...]
    c1 = m_ref[...] * ew
    c2 = jnp.sign(c1) * jnp.sqrt(jnp.abs(c1))
    c3 = jnp.maximum(c2 + eb, 0.0)
    out2 = _dot(c3, wEoA_ref[...]) + bEoA_ref[...]
    oe_ref[...] = out2[:, :DI]
    sc = jnp.clip(out2[:, DI:], -CLAMP, CLAMP)
    ex_ref[...] = jnp.exp(sc)


def _edge(cb, m, wEwb, bEb2, wEoA, bEoA2):
    nblk = E // _BE
    return pl.pallas_call(
        _edge_body,
        grid=(nblk,),
        in_specs=[
            pl.BlockSpec((_BE, HID), lambda i: (i, 0)),
            pl.BlockSpec((_BE, DI), lambda i: (i, 0)),
            pl.BlockSpec((HID, 2 * DI), lambda i: (0, 0)),
            pl.BlockSpec((1, DI), lambda i: (0, 0)),
            pl.BlockSpec((DI, DI + HEADS), lambda i: (0, 0)),
            pl.BlockSpec((1, DI + HEADS), lambda i: (0, 0)),
        ],
        out_specs=[
            pl.BlockSpec((_BE, DI), lambda i: (i, 0)),
            pl.BlockSpec((_BE, HEADS), lambda i: (i, 0)),
        ],
        out_shape=[
            jax.ShapeDtypeStruct((E, DI), jnp.float32),
            jax.ShapeDtypeStruct((E, HEADS), jnp.float32),
        ],
    )(cb, m, wEwb, bEb2, wEoA, bEoA2)


_CH3 = 1000


def _ssum_kernel(dst, exf, out, idx_v, ex_v, acc_v, sem1, sem2):
    cid = lax.axis_index("c")
    sid = lax.axis_index("s")
    wid = sid * NC + cid
    zero16 = jnp.zeros((16,), jnp.float32)
    iota = lax.iota(jnp.int32, 16)
    mask8 = iota < 8

    def zstep(j, carry):
        acc_v[pl.ds(j * 16, 16)] = zero16
        return carry

    lax.fori_loop(0, NF // 16, zstep, 0)
    base = wid * EPW

    def chunk(i, carry):
        off = base + i * _CH3
        d1 = pltpu.async_copy(dst.at[pl.ds(off, _CH3)], idx_v, sem1)
        d2 = pltpu.async_copy(exf.at[pl.ds(off * HEADS, _CH3 * HEADS)],
                              ex_v.at[pl.ds(0, _CH3 * HEADS)], sem2)
        d1.wait()
        d2.wait()

        def estep(j, c2):
            d16 = plsc.load_gather(idx_v, [jnp.full((16,), j, jnp.int32)])
            a16 = d16 * HEADS + iota
            x16 = ex_v[pl.ds(j * HEADS, 16)]
            plsc.addupdate_scatter(acc_v, [a16], x16, mask=mask8)
            return c2

        lax.fori_loop(0, _CH3, estep, 0)
        return carry

    lax.fori_loop(0, EPW // _CH3, chunk, 0)
    pltpu.sync_copy(acc_v, out.at[wid])


@functools.lru_cache(maxsize=None)
def _get_ssum_call():
    return functools.partial(
        pl.kernel,
        out_type=jax.ShapeDtypeStruct((NW, NF), jnp.float32),
        mesh=plsc.VectorSubcoreMesh(**_MESH),
        compiler_params=pltpu.CompilerParams(needs_layout_passes=False),
        scratch_types=[
            pltpu.VMEM((_CH3,), jnp.int32),
            pltpu.VMEM((_CH3 * HEADS + 16,), jnp.float32),
            pltpu.VMEM((NF,), jnp.float32),
            pltpu.SemaphoreType.DMA,
            pltpu.SemaphoreType.DMA,
        ],
    )(_ssum_kernel)


def _recip_body(pp_ref, p2_ref, out_ref):
    s = jnp.sum(pp_ref[...], axis=0)
    out_ref[...] = _dot(1.0 / (s + 1e-16), p2_ref[...])


def _recip(pp, p2):
    return pl.pallas_call(
        _recip_body,
        out_shape=jax.ShapeDtypeStruct((NF // HID, 16 * HID), jnp.float32),
    )(pp, p2)


_CH4 = 80


def _agg_kernel(dst, src, exf, oe, vp, zz, out,
                idxd_v, idxs_v, ex_v, oe_v, ct_v, sem1, sem2, sem3, acc_sh):
    cid = lax.axis_index("c")
    sid = lax.axis_index("s")
    wid = sid * NC + cid
    rslice = pl.ds(sid * TPD, TPD)
    pltpu.sync_copy(zz.at[rslice], acc_sh.at[rslice])
    plsc.subcore_barrier()
    iota = lax.iota(jnp.int32, 16)
    half = iota // HEADS
    base = wid * EPW

    def chunk(i, carry):
        off = base + i * _CH4
        d1 = pltpu.async_copy(dst.at[pl.ds(off, _CH4)], idxd_v, sem1)
        d2 = pltpu.async_copy(src.at[pl.ds(off, _CH4)], idxs_v, sem2)
        d1.wait()
        d2.wait()
        d3 = pltpu.async_copy(vp.at[idxs_v], ct_v, sem1)
        d4 = pltpu.async_copy(oe.at[pl.ds(off, _CH4)], oe_v, sem2)
        d5 = pltpu.async_copy(exf.at[pl.ds(off * HEADS, _CH4 * HEADS)],
                              ex_v.at[pl.ds(0, _CH4 * HEADS)], sem3)
        d3.wait()
        d4.wait()
        d5.wait()

        def row(r, c2):
            e16 = jnp.full((16,), r * HEADS, jnp.int32)
            for cc in range(DI // 16):
                w = plsc.load_gather(ex_v, [e16 + (cc * 2 + half)])
                ss = pl.ds(cc * 16, 16)
                s2 = pl.ds(DI + cc * 16, 16)
                ct_v[r, ss] = ct_v[r, ss] * w
                ct_v[r, s2] = oe_v[r, ss] * w
            return c2

        lax.fori_loop(0, _CH4, row, 0)
        pltpu.sync_copy(ct_v, acc_sh.at[idxd_v], add=True)
        return carry

    lax.fori_loop(0, EPW // _CH4, chunk, 0)
    plsc.subcore_barrier()
    pltpu.sync_copy(acc_sh.at[rslice], out.at[cid, rslice])


@functools.lru_cache(maxsize=None)
def _get_agg_call():
    return functools.partial(
        pl.kernel,
        out_type=jax.ShapeDtypeStruct((NC, NPAD, 2 * DI), jnp.float32),
        mesh=plsc.VectorSubcoreMesh(**_MESH),
        compiler_params=pltpu.CompilerParams(needs_layout_passes=False),
        scratch_types=[
            pltpu.VMEM((_CH4,), jnp.int32),
            pltpu.VMEM((_CH4,), jnp.int32),
            pltpu.VMEM((_CH4 * HEADS + 16,), jnp.float32),
            pltpu.VMEM((_CH4, DI), jnp.float32),
            pltpu.VMEM((_CH4, 2 * DI), jnp.float32),
            pltpu.SemaphoreType.DMA,
            pltpu.SemaphoreType.DMA,
            pltpu.SemaphoreType.DMA,
            pltpu.VMEM_SHARED((NPAD, 2 * DI), jnp.float32),
        ],
    )(_agg_kernel)


def _final_body(p_ref, r_ref, b2_ref, out_ref):
    s = (p_ref[0] + p_ref[1]) * r_ref[...]
    sn = s[:N]
    out_ref[...] = sn[:, :DI] + _dot(sn[:, DI:], b2_ref[...])


def _final(parts, rsum, b2):
    return pl.pallas_call(
        _final_body,
        out_shape=jax.ShapeDtypeStruct((N, DI), jnp.float32),
    )(parts, rsum, b2)


def kernel(x, rrwp_index, rrwp_conn, WQ, WK, WV, WEw, WEb, bEb, WEo, bEo,
           Aw, BW):
    f32 = jnp.float32
    dst = rrwp_index[0].astype(jnp.int32)
    src = rrwp_index[1].astype(jnp.int32)

    w3 = jnp.concatenate([WQ, WK, WV], axis=1)
    wEwb = jnp.concatenate([WEw, WEb], axis=1)
    bEb2 = bEb.reshape(1, DI)
    i64 = jnp.arange(DI)
    A2 = jnp.zeros((DI, HEADS), f32).at[i64, i64 // DIM].set(
        Aw[:, :, 0].T.reshape(DI))
    wEoA = jnp.concatenate([WEo, WEo @ A2], axis=1)
    bEoA = jnp.concatenate([bEo, bEo @ A2]).reshape(1, DI + HEADS)
    B2 = jnp.zeros((DI, DI), f32).at[
        i64[:, None],
        (i64[:, None] // DIM) * DIM + jnp.arange(DIM)[None, :]].set(
        BW.transpose(1, 0, 2).reshape(DI, DIM))

    zz128 = jnp.zeros((NPAD, 2 * DI), f32)

    qk, vp = _qkv(x, w3)
    msg1 = _get_msg_call()(qk, dst, src)
    oe, ex8 = _edge(rrwp_conn, msg1, wEwb, bEb2, wEoA, bEoA)
    exf = ex8.reshape(E * HEADS)
    a2048 = jnp.arange(16 * HID)
    P2 = jnp.zeros((HID, 16 * HID), f32).at[
        HEADS * (a2048 // HID) + ((a2048 % HID) % DI) // HEADS, a2048].set(1.0)
    ssum_p = _get_ssum_call()(dst, exf)
    rsum = _recip(ssum_p.reshape(NW, NF // HID, HID), P2).reshape(NPAD, HID)
    parts = _get_agg_call()(dst, src, exf, oe, vp, zz128)
    h_out = _final(parts, rsum, B2)
    return (h_out, oe)
